# baseline (speedup 1.0000x reference)
import functools

import jax
import jax.numpy as jnp
from jax import lax
from jax.experimental import pallas as pl
from jax.experimental.pallas import tpu as pltpu
from jax.experimental.pallas import tpu_sc as plsc

B, N, C, D = 128, 900, 81, 256
K = 50
KP = 56
KL = 128
CT = 128
NEG = -3.4e38

NC_, NS_ = 2, 16
NW = NC_ * NS_
IPW = B // NW


def _scores_body(cls_ref, s_ref):
    x = cls_ref[...]
    M = jnp.max(x, axis=0)
    e = jnp.exp(x - M[None])
    S = jnp.sum(e, axis=0)
    m80 = jnp.max(x[: C - 1], axis=0)
    s_ref[...] = jnp.exp(m80 - M) / S


_scores = pl.pallas_call(
    _scores_body,
    grid=(pl.cdiv(N, CT),),
    in_specs=[pl.BlockSpec((C, CT, B), lambda i: (0, i, 0))],
    out_specs=pl.BlockSpec((CT, B), lambda i: (i, 0)),
    out_shape=jax.ShapeDtypeStruct((N, B), jnp.float32),
)


def _topk_body(s_ref, idx_ref):
    s = s_ref[...]
    row = lax.broadcasted_iota(jnp.int32, (N, B), 0)
    subi = lax.broadcasted_iota(jnp.int32, (KL, B), 0)
    lanei = lax.broadcasted_iota(jnp.int32, (KL, B), 1)

    def step(k, carry):
        s, acc = carry
        m = jnp.max(s, axis=0)
        cand = jnp.where(s == m[None], row, N)
        idx = jnp.min(cand, axis=0)
        acc = jnp.where(subi == k, idx[None] * B + lanei, acc)
        s = jnp.where(row == idx[None], NEG, s)
        return s, acc

    _, acc = lax.fori_loop(0, K, step, (s, jnp.zeros((KL, B), jnp.int32)))
    idx_ref[...] = acc


_topk = pl.pallas_call(
    _topk_body,
    in_specs=[pl.BlockSpec((N, B), lambda: (0, 0))],
    out_specs=pl.BlockSpec((KL, B), lambda: (0, 0)),
    out_shape=jax.ShapeDtypeStruct((KL, B), jnp.int32),
    grid=(),
)


@functools.cache
def _make_gather():
    @functools.partial(
        pl.kernel,
        mesh=plsc.VectorSubcoreMesh(core_axis_name="c", subcore_axis_name="s"),
        out_type=jax.ShapeDtypeStruct((B, KP, D), jnp.float32),
        scratch_types=[
            pltpu.VMEM((IPW, KL), jnp.int32),
            pltpu.VMEM((IPW, KP, D), jnp.float32),
            pltpu.SemaphoreType.DMA,
            pltpu.SemaphoreType.DMA,
        ],
    )
    def _gather_rows(table_hbm, idx_hbm, out_hbm, idx_v, rows_v, gsem, osem):
        wid = lax.axis_index("s") * NC_ + lax.axis_index("c")
        b0 = wid * IPW
        pltpu.sync_copy(idx_hbm.at[pl.ds(b0, IPW)], idx_v)
        gathers = [
            pltpu.async_copy(
                table_hbm.at[idx_v.at[i, pl.ds(0, KP)]], rows_v.at[i], gsem
            )
            for i in range(IPW)
        ]
        writes = []
        for i in range(IPW):
            gathers[i].wait()
            writes.append(
                pltpu.async_copy(rows_v.at[i], out_hbm.at[b0 + i], osem)
            )
        for w in writes:
            w.wait()

    return _gather_rows


def kernel(reg, cls_score):
    cls_t = jnp.transpose(cls_score, (2, 1, 0))
    table = jnp.transpose(reg, (1, 0, 2)).reshape(N * B, D)
    s = _scores(cls_t)
    idx_kb = _topk(s)
    idx = idx_kb.T
    out = _make_gather()(table, idx)
    return out[:, :K, :]


# speedup vs baseline: 1.0304x; 1.0304x over previous
import functools

import jax
import jax.numpy as jnp
from jax import lax
from jax.experimental import pallas as pl
from jax.experimental.pallas import tpu as pltpu
from jax.experimental.pallas import tpu_sc as plsc

B, N, C, D = 128, 900, 81, 256
K = 50
KP = 56
KL = 128
CT = 128
NEG = -3.4e38

NC_, NS_ = 2, 16
NW = NC_ * NS_
IPW = B // NW


def _scores_body(cls_ref, s_ref):
    x = cls_ref[...]
    M = jnp.max(x, axis=0)
    e = jnp.exp(x - M[None])
    S = jnp.sum(e, axis=0)
    m80 = jnp.max(x[: C - 1], axis=0)
    s_ref[...] = jnp.exp(m80 - M) / S


_scores = pl.pallas_call(
    _scores_body,
    grid=(pl.cdiv(N, CT),),
    in_specs=[pl.BlockSpec((C, CT, B), lambda i: (0, i, 0))],
    out_specs=pl.BlockSpec((CT, B), lambda i: (i, 0)),
    out_shape=jax.ShapeDtypeStruct((N, B), jnp.float32),
)


def _topk_body(s_ref, idx_ref):
    s = s_ref[...]
    row = lax.broadcasted_iota(jnp.int32, (N, B), 0)
    subi = lax.broadcasted_iota(jnp.int32, (KL, B), 0)
    lanei = lax.broadcasted_iota(jnp.int32, (KL, B), 1)

    def step(k, carry):
        s, acc = carry
        m = jnp.max(s, axis=0)
        cand = jnp.where(s == m[None], row, N)
        idx = jnp.min(cand, axis=0)
        acc = jnp.where(subi == k, idx[None] * B + lanei, acc)
        s = jnp.where(row == idx[None], NEG, s)
        return s, acc

    _, acc = lax.fori_loop(0, K, step, (s, jnp.zeros((KL, B), jnp.int32)))
    idx_ref[...] = acc


_topk = pl.pallas_call(
    _topk_body,
    in_specs=[pl.BlockSpec((N, B), lambda: (0, 0))],
    out_specs=pl.BlockSpec((KL, B), lambda: (0, 0)),
    out_shape=jax.ShapeDtypeStruct((KL, B), jnp.int32),
    grid=(),
)


@functools.cache
def _make_gather():
    @functools.partial(
        pl.kernel,
        mesh=plsc.VectorSubcoreMesh(core_axis_name="c", subcore_axis_name="s"),
        out_type=jax.ShapeDtypeStruct((B, KP, D), jnp.float32),
        scratch_types=[
            pltpu.VMEM((IPW, KL), jnp.int32),
            pltpu.VMEM((IPW, KP, D), jnp.float32),
            pltpu.SemaphoreType.DMA,
            pltpu.SemaphoreType.DMA,
            pltpu.SemaphoreType.DMA,
        ],
    )
    def _gather_rows(table_hbm, idx_hbm, out_hbm, idx_v, rows_v, gsem, dsem,
                     osem):
        wid = lax.axis_index("s") * NC_ + lax.axis_index("c")
        b0 = wid * IPW
        pltpu.sync_copy(idx_hbm.at[pl.ds(b0, IPW)], idx_v)
        streams = [
            pltpu.async_copy(
                table_hbm.at[idx_v.at[i, pl.ds(0, 32)]],
                rows_v.at[i, pl.ds(0, 32)], gsem
            )
            for i in range(IPW)
        ]
        dmas = []
        for i in range(IPW):
            for c, w in ((32, 16), (48, 8)):
                vec = idx_v[i, pl.ds(c, 16)]
                for j in range(w):
                    dmas.append(pltpu.async_copy(
                        table_hbm.at[pl.ds(vec[j], 1)],
                        rows_v.at[i, pl.ds(c + j, 1)], dsem))
        for d in dmas:
            d.wait()
        writes = []
        for i in range(IPW):
            streams[i].wait()
            writes.append(
                pltpu.async_copy(rows_v.at[i], out_hbm.at[b0 + i], osem)
            )
        for w in writes:
            w.wait()

    return _gather_rows


def kernel(reg, cls_score):
    cls_t = jnp.transpose(cls_score, (2, 1, 0))
    table = jnp.transpose(reg, (1, 0, 2)).reshape(N * B, D)
    s = _scores(cls_t)
    idx_kb = _topk(s)
    idx = idx_kb.T
    out = _make_gather()(table, idx)
    return out[:, :K, :]
